# Initial kernel scaffold; baseline (speedup 1.0000x reference)
import jax
import jax.numpy as jnp
from jax.experimental import pallas as pl


def kernel(idx, table):
    raise NotImplementedError("write your pallas kernel here")


# speedup vs baseline: 1.0320x; 1.0320x over previous
import functools

import jax
import jax.numpy as jnp
from jax import lax
from jax.experimental import pallas as pl
from jax.experimental.pallas import tpu as pltpu
from jax.experimental.pallas import tpu_sc as plsc

VOCAB = 1000
B_TOT = 1024 * 50
NC, NS = 2, 16
NW = NC * NS
BPW = B_TOT // NW
CH = 64
NCH = BPW // CH


@functools.partial(
    pl.kernel,
    out_type=jax.ShapeDtypeStruct((B_TOT, VOCAB), jnp.float32),
    mesh=plsc.VectorSubcoreMesh(core_axis_name="c", subcore_axis_name="s"),
    scratch_types=[
        pltpu.VMEM((BPW,), jnp.int32),
        pltpu.VMEM((2, CH, VOCAB), jnp.float32),
        pltpu.SemaphoreType.DMA,
    ],
    compiler_params=pltpu.CompilerParams(use_tc_tiling_on_sc=False),
)
def _sc_gather(idx_hbm, table_hbm, out_hbm, idx_v, rows_v, sem):
    wid = lax.axis_index("s") * NC + lax.axis_index("c")
    base = wid * BPW
    pltpu.sync_copy(idx_hbm.at[pl.ds(base, BPW)], idx_v)

    copies = [None] * NCH
    for c in range(NCH):
        copies[c] = pltpu.async_copy(
            table_hbm.at[idx_v.at[pl.ds(c * CH, CH)]], rows_v.at[c % 2], sem
        )
        if c >= 1:
            copies[c - 1].wait()
            pltpu.sync_copy(
                rows_v.at[(c - 1) % 2],
                out_hbm.at[pl.ds(base + (c - 1) * CH, CH)],
            )
    copies[NCH - 1].wait()
    pltpu.sync_copy(
        rows_v.at[(NCH - 1) % 2],
        out_hbm.at[pl.ds(base + (NCH - 1) * CH, CH)],
    )


def kernel(idx, table):
    b, s = idx.shape
    idx_flat = idx.reshape(-1).astype(jnp.int32)
    out = _sc_gather(idx_flat, table)
    return out.reshape(b, s, VOCAB)
